# baseline (speedup 1.0000x reference)
import functools

import jax
import jax.numpy as jnp
from jax.experimental import pallas as pl
from jax.experimental.pallas import tpu as pltpu

_Q = 1024
_D = 16
_KBLK = 2048


def _nn_kernel(nsteps, x_ref, pt_ref, out_ref, minv_ref, mini_ref):
    step = pl.program_id(0)
    x = x_ref[...]
    pt = pt_ref[...]

    dot = jax.lax.dot_general(
        x, pt, (((1,), (0,)), ((), ())),
        preferred_element_type=jnp.float32,
        precision=jax.lax.Precision.DEFAULT,
    )
    x2 = jnp.sum(x * x, axis=1, keepdims=True)
    p2 = jnp.sum(pt * pt, axis=0, keepdims=True)
    d2 = jnp.maximum(x2 + p2 - 2.0 * dot, 0.0)
    dist = jnp.sqrt(d2)

    bmin = jnp.min(dist, axis=1, keepdims=True)
    lane = jax.lax.broadcasted_iota(jnp.int32, dist.shape, 1)
    bidx = jnp.min(jnp.where(dist == bmin, lane, _KBLK),
                   axis=1, keepdims=True) + step * _KBLK

    @pl.when(step == 0)
    def _():
        minv_ref[...] = bmin
        mini_ref[...] = bidx

    @pl.when(step > 0)
    def _():
        better = bmin < minv_ref[...]
        minv_ref[...] = jnp.where(better, bmin, minv_ref[...])
        mini_ref[...] = jnp.where(better, bidx, mini_ref[...])

    @pl.when(step == nsteps - 1)
    def _():
        out_ref[...] = mini_ref[...]


def kernel(X, Phi):
    k = Phi.shape[0]
    nsteps = -(-k // _KBLK)
    kpad = nsteps * _KBLK
    phi_t = jnp.pad(Phi.T, ((0, 0), (0, kpad - k)), constant_values=1e17)

    out = pl.pallas_call(
        functools.partial(_nn_kernel, nsteps),
        grid=(nsteps,),
        in_specs=[
            pl.BlockSpec((_Q, _D), lambda i: (0, 0)),
            pl.BlockSpec((_D, _KBLK), lambda i: (0, i)),
        ],
        out_specs=pl.BlockSpec((_Q, 1), lambda i: (0, 0)),
        out_shape=jax.ShapeDtypeStruct((_Q, 1), jnp.int32),
        scratch_shapes=[
            pltpu.VMEM((_Q, 1), jnp.float32),
            pltpu.VMEM((_Q, 1), jnp.int32),
        ],
    )(X, phi_t)
    return out.reshape(-1)


# speedup vs baseline: 1.0954x; 1.0954x over previous
import functools

import jax
import jax.numpy as jnp
from jax.experimental import pallas as pl
from jax.experimental.pallas import tpu as pltpu

_Q = 1024
_D = 16
_KBLK = 2048
_BIG = 2**30


def _tie_threshold(m):
    s = jnp.sqrt(m)
    base = jax.lax.bitcast_convert_type(s * s, jnp.int32)
    t = m
    for off in range(-2, 6):
        ci = jnp.maximum(base + off, 0)
        c = jax.lax.bitcast_convert_type(ci, jnp.float32)
        t = jnp.where(jnp.sqrt(c) == s, jnp.maximum(t, c), t)
    return t


def _nn_kernel(nsteps, x_ref, pt_ref, out_ref, minv_ref, mini_ref, thr_ref):
    phase = pl.program_id(0)
    step = pl.program_id(1)
    x = x_ref[...]
    pt = pt_ref[...]

    dot = jax.lax.dot_general(
        x, pt, (((1,), (0,)), ((), ())),
        preferred_element_type=jnp.float32,
        precision=jax.lax.Precision.DEFAULT,
    )
    x2 = jnp.sum(x * x, axis=1, keepdims=True)
    p2 = jnp.sum(pt * pt, axis=0, keepdims=True)
    v = x2 + p2 - 2.0 * dot

    @pl.when(phase == 0)
    def _():
        bmin = jnp.min(v, axis=1, keepdims=True)

        @pl.when(step == 0)
        def _():
            minv_ref[...] = bmin

        @pl.when(step > 0)
        def _():
            minv_ref[...] = jnp.minimum(minv_ref[...], bmin)

        @pl.when(step == nsteps - 1)
        def _():
            thr_ref[...] = _tie_threshold(jnp.maximum(minv_ref[...], 0.0))

    @pl.when(phase == 1)
    def _():
        lane = jax.lax.broadcasted_iota(jnp.int32, v.shape, 1)
        hit = jnp.where(v <= thr_ref[...], lane, _BIG)
        bidx = jnp.min(hit, axis=1, keepdims=True) + step * _KBLK

        @pl.when(step == 0)
        def _():
            mini_ref[...] = bidx

        @pl.when(step > 0)
        def _():
            mini_ref[...] = jnp.minimum(mini_ref[...], bidx)

        @pl.when(step == nsteps - 1)
        def _():
            out_ref[...] = mini_ref[...]


def kernel(X, Phi):
    k = Phi.shape[0]
    nsteps = -(-k // _KBLK)
    kpad = nsteps * _KBLK
    phi_t = jnp.pad(Phi.T, ((0, 0), (0, kpad - k)), constant_values=1e17)

    out = pl.pallas_call(
        functools.partial(_nn_kernel, nsteps),
        grid=(2, nsteps),
        in_specs=[
            pl.BlockSpec((_Q, _D), lambda p, j: (0, 0)),
            pl.BlockSpec((_D, _KBLK), lambda p, j: (0, j)),
        ],
        out_specs=pl.BlockSpec((_Q, 1), lambda p, j: (0, 0)),
        out_shape=jax.ShapeDtypeStruct((_Q, 1), jnp.int32),
        scratch_shapes=[
            pltpu.VMEM((_Q, 1), jnp.float32),
            pltpu.VMEM((_Q, 1), jnp.int32),
            pltpu.VMEM((_Q, 1), jnp.float32),
        ],
    )(X, phi_t)
    return out.reshape(-1)


# speedup vs baseline: 1.1454x; 1.0457x over previous
import functools

import jax
import jax.numpy as jnp
from jax.experimental import pallas as pl
from jax.experimental.pallas import tpu as pltpu

_Q = 1024
_D = 16
_KBLK = 2048
_C = 64
_BIGF = float(2 ** 25)


def _dist_block(x, pt):
    dot = jax.lax.dot_general(
        x, pt, (((1,), (0,)), ((), ())),
        preferred_element_type=jnp.float32,
        precision=jax.lax.Precision.DEFAULT,
    )
    x2 = jnp.sum(x * x, axis=1, keepdims=True)
    p2 = jnp.sum(pt * pt, axis=0, keepdims=True)
    return x2 + p2 - 2.0 * dot


def _scan_kernel(x_ref, pt_ref, out_ref):
    v = _dist_block(x_ref[...], pt_ref[...])
    out_ref[0] = jnp.min(v, axis=1, keepdims=True)


def _first_hit(v, t):
    lane = jax.lax.broadcasted_iota(jnp.int32, v.shape, 1).astype(jnp.float32)
    hit = jnp.where(v <= t, lane, _BIGF)
    return jnp.min(hit, axis=1, keepdims=True)


def _rescore_kernel(xg_ref, pt_ref, tg_ref, out_ref):
    b = pl.program_id(0)
    xg = xg_ref[0]
    tg = tg_ref[0]
    v = _dist_block(xg, pt_ref[...])
    out_ref[0] = _first_hit(v, tg) + b * float(_KBLK)


def _sweep_kernel(nsteps, x_ref, pt_ref, t_ref, out_ref, mini_ref):
    step = pl.program_id(0)
    v = _dist_block(x_ref[...], pt_ref[...])
    bidx = _first_hit(v, t_ref[...]) + step * float(_KBLK)

    @pl.when(step == 0)
    def _():
        mini_ref[...] = bidx

    @pl.when(step > 0)
    def _():
        mini_ref[...] = jnp.minimum(mini_ref[...], bidx)

    @pl.when(step == nsteps - 1)
    def _():
        out_ref[...] = mini_ref[...].astype(jnp.int32)


def _tie_threshold(m):
    s = jnp.sqrt(m)
    base = jax.lax.bitcast_convert_type(s * s, jnp.int32)
    t = m
    for off in range(-2, 8):
        c = jax.lax.bitcast_convert_type(jnp.maximum(base + off, 0), jnp.float32)
        t = jnp.where(jnp.sqrt(c) == s, jnp.maximum(t, c), t)
    return t


def kernel(X, Phi):
    k = Phi.shape[0]
    nsteps = -(-k // _KBLK)
    kpad = nsteps * _KBLK
    phi_t = jnp.pad(Phi.T, ((0, 0), (0, kpad - k)), constant_values=1e17)

    blkmin = pl.pallas_call(
        _scan_kernel,
        grid=(nsteps,),
        in_specs=[
            pl.BlockSpec((_Q, _D), lambda j: (0, 0)),
            pl.BlockSpec((_D, _KBLK), lambda j: (0, j)),
        ],
        out_specs=pl.BlockSpec((1, _Q, 1), lambda j: (j, 0, 0)),
        out_shape=jax.ShapeDtypeStruct((nsteps, _Q, 1), jnp.float32),
    )(X, phi_t)
    blkmin = blkmin.reshape(nsteps, _Q).T

    m = jnp.maximum(jnp.min(blkmin, axis=1), 0.0)
    t = _tie_threshold(m)
    cand = blkmin <= t[:, None]
    counts = jnp.sum(cand.astype(jnp.int32), axis=0)
    overflow = jnp.any(counts > _C)

    enc = jnp.where(cand, _Q - jnp.arange(_Q, dtype=jnp.int32)[:, None], 0)
    vals = jax.lax.top_k(enc.T, _C)[0]
    lists = jnp.where(vals > 0, _Q - vals, _Q)

    x_pad = jnp.concatenate([X, jnp.zeros((1, _D), X.dtype)], axis=0)
    t_pad = jnp.concatenate([t, jnp.full((1,), -jnp.inf, t.dtype)], axis=0)
    xg = x_pad[lists]
    tg = t_pad[lists][..., None]

    def fast_path(_):
        idx = pl.pallas_call(
            _rescore_kernel,
            grid=(nsteps,),
            in_specs=[
                pl.BlockSpec((1, _C, _D), lambda j: (j, 0, 0)),
                pl.BlockSpec((_D, _KBLK), lambda j: (0, j)),
                pl.BlockSpec((1, _C, 1), lambda j: (j, 0, 0)),
            ],
            out_specs=pl.BlockSpec((1, _C, 1), lambda j: (j, 0, 0)),
            out_shape=jax.ShapeDtypeStruct((nsteps, _C, 1), jnp.float32),
        )(xg, phi_t, tg)
        idx = idx.reshape(nsteps * _C).astype(jnp.int32)
        out = jnp.full((_Q + 1,), 2 ** 30, jnp.int32)
        out = out.at[lists.reshape(-1)].min(idx, mode="drop")
        return out[:_Q]

    def slow_path(_):
        out = pl.pallas_call(
            functools.partial(_sweep_kernel, nsteps),
            grid=(nsteps,),
            in_specs=[
                pl.BlockSpec((_Q, _D), lambda j: (0, 0)),
                pl.BlockSpec((_D, _KBLK), lambda j: (0, j)),
                pl.BlockSpec((_Q, 1), lambda j: (0, 0)),
            ],
            out_specs=pl.BlockSpec((_Q, 1), lambda j: (0, 0)),
            out_shape=jax.ShapeDtypeStruct((_Q, 1), jnp.int32),
            scratch_shapes=[pltpu.VMEM((_Q, 1), jnp.float32)],
        )(X, phi_t, t[:, None])
        return out.reshape(-1)

    return jax.lax.cond(overflow, slow_path, fast_path, operand=None)


# speedup vs baseline: 1.2122x; 1.0583x over previous
import functools

import jax
import jax.numpy as jnp
from jax.experimental import pallas as pl
from jax.experimental.pallas import tpu as pltpu

_Q = 1024
_D = 16
_KBLK = 2048
_BIGF = float(2 ** 25)


def _dist_block(x, pt):
    dot = jax.lax.dot_general(
        x, pt, (((1,), (0,)), ((), ())),
        preferred_element_type=jnp.float32,
        precision=jax.lax.Precision.DEFAULT,
    )
    x2 = jnp.sum(x * x, axis=1, keepdims=True)
    p2 = jnp.sum(pt * pt, axis=0, keepdims=True)
    return x2 + p2 - 2.0 * dot


def _first_hit(v, t):
    lane = jax.lax.broadcasted_iota(jnp.int32, v.shape, 1).astype(jnp.float32)
    hit = jnp.where(v <= t, lane, _BIGF)
    return jnp.min(hit, axis=1, keepdims=True)


def _tie_threshold(m):
    s = jnp.sqrt(m)
    base = jax.lax.bitcast_convert_type(s * s, jnp.int32)
    t = m
    for off in range(-2, 8):
        c = jax.lax.bitcast_convert_type(jnp.maximum(base + off, 0), jnp.float32)
        t = jnp.where(jnp.sqrt(c) == s, jnp.maximum(t, c), t)
    return t


def _nn_kernel(nsteps, x_ref, pt_ref, out_ref, minv_ref, mini_ref, thr_ref):
    phase = pl.program_id(0)
    step = pl.program_id(1)
    v = _dist_block(x_ref[...], pt_ref[...])

    @pl.when(phase == 0)
    def _():
        bmin = jnp.min(v, axis=1, keepdims=True)

        @pl.when(step == 0)
        def _():
            minv_ref[...] = bmin

        @pl.when(step > 0)
        def _():
            minv_ref[...] = jnp.minimum(minv_ref[...], bmin)

        @pl.when(step == nsteps - 1)
        def _():
            thr_ref[...] = _tie_threshold(jnp.maximum(minv_ref[...], 0.0))

    @pl.when(phase == 1)
    def _():
        bidx = _first_hit(v, thr_ref[...]) + step * float(_KBLK)

        @pl.when(step == 0)
        def _():
            mini_ref[...] = bidx

        @pl.when(step > 0)
        def _():
            mini_ref[...] = jnp.minimum(mini_ref[...], bidx)

        @pl.when(step == nsteps - 1)
        def _():
            out_ref[...] = mini_ref[...].astype(jnp.int32)


def kernel(X, Phi):
    k = Phi.shape[0]
    nsteps = -(-k // _KBLK)
    kpad = nsteps * _KBLK
    phi_t = jnp.pad(Phi.T, ((0, 0), (0, kpad - k)), constant_values=1e17)

    out = pl.pallas_call(
        functools.partial(_nn_kernel, nsteps),
        grid=(2, nsteps),
        in_specs=[
            pl.BlockSpec((_Q, _D), lambda p, j: (0, 0)),
            pl.BlockSpec((_D, _KBLK), lambda p, j: (0, j)),
        ],
        out_specs=pl.BlockSpec((_Q, 1), lambda p, j: (0, 0)),
        out_shape=jax.ShapeDtypeStruct((_Q, 1), jnp.int32),
        scratch_shapes=[
            pltpu.VMEM((_Q, 1), jnp.float32),
            pltpu.VMEM((_Q, 1), jnp.float32),
            pltpu.VMEM((_Q, 1), jnp.float32),
        ],
    )(X, phi_t)
    return out.reshape(-1)


# speedup vs baseline: 1.2948x; 1.0682x over previous
import functools

import jax
import jax.numpy as jnp
from jax.experimental import pallas as pl
from jax.experimental.pallas import tpu as pltpu

_Q = 1024
_D = 16
_KBLK = 4096
_BIGF = float(2 ** 25)


def _dist_block(x, pt):
    dot = jax.lax.dot_general(
        x, pt, (((1,), (0,)), ((), ())),
        preferred_element_type=jnp.float32,
        precision=jax.lax.Precision.DEFAULT,
    )
    x2 = jnp.sum(x * x, axis=1, keepdims=True)
    p2 = jnp.sum(pt * pt, axis=0, keepdims=True)
    return x2 + p2 - 2.0 * dot


def _first_hit(v, t):
    lane = jax.lax.broadcasted_iota(jnp.int32, v.shape, 1).astype(jnp.float32)
    hit = jnp.where(v <= t, lane, _BIGF)
    return jnp.min(hit, axis=1, keepdims=True)


def _tie_threshold(m):
    s = jnp.sqrt(m)
    base = jax.lax.bitcast_convert_type(s * s, jnp.int32)
    t = m
    for off in range(-2, 8):
        c = jax.lax.bitcast_convert_type(jnp.maximum(base + off, 0), jnp.float32)
        t = jnp.where(jnp.sqrt(c) == s, jnp.maximum(t, c), t)
    return t


def _nn_kernel(nsteps, x_ref, pt_ref, out_ref, minv_ref, mini_ref, thr_ref):
    phase = pl.program_id(0)
    step = pl.program_id(1)
    v = _dist_block(x_ref[...], pt_ref[...])

    @pl.when(phase == 0)
    def _():
        bmin = jnp.min(v, axis=1, keepdims=True)

        @pl.when(step == 0)
        def _():
            minv_ref[...] = bmin

        @pl.when(step > 0)
        def _():
            minv_ref[...] = jnp.minimum(minv_ref[...], bmin)

        @pl.when(step == nsteps - 1)
        def _():
            thr_ref[...] = _tie_threshold(jnp.maximum(minv_ref[...], 0.0))

    @pl.when(phase == 1)
    def _():
        bidx = _first_hit(v, thr_ref[...]) + step * float(_KBLK)

        @pl.when(step == 0)
        def _():
            mini_ref[...] = bidx

        @pl.when(step > 0)
        def _():
            mini_ref[...] = jnp.minimum(mini_ref[...], bidx)

        @pl.when(step == nsteps - 1)
        def _():
            out_ref[...] = mini_ref[...].astype(jnp.int32)


def kernel(X, Phi):
    k = Phi.shape[0]
    nsteps = -(-k // _KBLK)
    kpad = nsteps * _KBLK
    phi_t = jnp.pad(Phi.T, ((0, 0), (0, kpad - k)), constant_values=1e17)

    out = pl.pallas_call(
        functools.partial(_nn_kernel, nsteps),
        grid=(2, nsteps),
        in_specs=[
            pl.BlockSpec((_Q, _D), lambda p, j: (0, 0)),
            pl.BlockSpec((_D, _KBLK), lambda p, j: (0, j)),
        ],
        out_specs=pl.BlockSpec((_Q, 1), lambda p, j: (0, 0)),
        out_shape=jax.ShapeDtypeStruct((_Q, 1), jnp.int32),
        scratch_shapes=[
            pltpu.VMEM((_Q, 1), jnp.float32),
            pltpu.VMEM((_Q, 1), jnp.float32),
            pltpu.VMEM((_Q, 1), jnp.float32),
        ],
    )(X, phi_t)
    return out.reshape(-1)


# speedup vs baseline: 1.2959x; 1.0009x over previous
import functools

import jax
import jax.numpy as jnp
from jax.experimental import pallas as pl
from jax.experimental.pallas import tpu as pltpu

_Q = 1024
_D = 16
_KBLK = 8192
_BIGF = float(2 ** 25)


def _dist_block(x, pt):
    dot = jax.lax.dot_general(
        x, pt, (((1,), (0,)), ((), ())),
        preferred_element_type=jnp.float32,
        precision=jax.lax.Precision.DEFAULT,
    )
    x2 = jnp.sum(x * x, axis=1, keepdims=True)
    p2 = jnp.sum(pt * pt, axis=0, keepdims=True)
    return x2 + p2 - 2.0 * dot


def _first_hit(v, t):
    lane = jax.lax.broadcasted_iota(jnp.int32, v.shape, 1).astype(jnp.float32)
    hit = jnp.where(v <= t, lane, _BIGF)
    return jnp.min(hit, axis=1, keepdims=True)


def _tie_threshold(m):
    s = jnp.sqrt(m)
    base = jax.lax.bitcast_convert_type(s * s, jnp.int32)
    t = m
    for off in range(-2, 8):
        c = jax.lax.bitcast_convert_type(jnp.maximum(base + off, 0), jnp.float32)
        t = jnp.where(jnp.sqrt(c) == s, jnp.maximum(t, c), t)
    return t


def _nn_kernel(nsteps, x_ref, pt_ref, out_ref, minv_ref, mini_ref, thr_ref):
    phase = pl.program_id(0)
    step = pl.program_id(1)
    v = _dist_block(x_ref[...], pt_ref[...])

    @pl.when(phase == 0)
    def _():
        bmin = jnp.min(v, axis=1, keepdims=True)

        @pl.when(step == 0)
        def _():
            minv_ref[...] = bmin

        @pl.when(step > 0)
        def _():
            minv_ref[...] = jnp.minimum(minv_ref[...], bmin)

        @pl.when(step == nsteps - 1)
        def _():
            thr_ref[...] = _tie_threshold(jnp.maximum(minv_ref[...], 0.0))

    @pl.when(phase == 1)
    def _():
        bidx = _first_hit(v, thr_ref[...]) + step * float(_KBLK)

        @pl.when(step == 0)
        def _():
            mini_ref[...] = bidx

        @pl.when(step > 0)
        def _():
            mini_ref[...] = jnp.minimum(mini_ref[...], bidx)

        @pl.when(step == nsteps - 1)
        def _():
            out_ref[...] = mini_ref[...].astype(jnp.int32)


def kernel(X, Phi):
    k = Phi.shape[0]
    nsteps = -(-k // _KBLK)
    kpad = nsteps * _KBLK
    phi_t = jnp.pad(Phi.T, ((0, 0), (0, kpad - k)), constant_values=1e17)

    out = pl.pallas_call(
        functools.partial(_nn_kernel, nsteps),
        grid=(2, nsteps),
        in_specs=[
            pl.BlockSpec((_Q, _D), lambda p, j: (0, 0)),
            pl.BlockSpec((_D, _KBLK), lambda p, j: (0, j)),
        ],
        out_specs=pl.BlockSpec((_Q, 1), lambda p, j: (0, 0)),
        out_shape=jax.ShapeDtypeStruct((_Q, 1), jnp.int32),
        scratch_shapes=[
            pltpu.VMEM((_Q, 1), jnp.float32),
            pltpu.VMEM((_Q, 1), jnp.float32),
            pltpu.VMEM((_Q, 1), jnp.float32),
        ],
    )(X, phi_t)
    return out.reshape(-1)
